# initial kernel scaffold (unmeasured)
import jax
import jax.numpy as jnp
from jax import lax
from jax.experimental import pallas as pl
from jax.experimental.pallas import tpu as pltpu

M, D = 8192, 2048
EPS = 1e-6


def _exchange(partial):

    def body(p_ref, out_ref, send_sem, recv_sem):
        my_x = lax.axis_index("x")
        my_y = lax.axis_index("y")
        my_z = lax.axis_index("z")
        nbr = (my_x, 1 - my_y, my_z)

        barrier = pltpu.get_barrier_semaphore()
        pl.semaphore_signal(
            barrier, inc=1, device_id=nbr, device_id_type=pl.DeviceIdType.MESH
        )
        pl.semaphore_wait(barrier, 1)

        rdma = pltpu.make_async_remote_copy(
            src_ref=p_ref,
            dst_ref=out_ref,
            send_sem=send_sem,
            recv_sem=recv_sem,
            device_id=nbr,
            device_id_type=pl.DeviceIdType.MESH,
        )
        rdma.start()
        rdma.wait()

    return pl.pallas_call(
        body,
        out_shape=jax.ShapeDtypeStruct((1, M, D), jnp.float32),
        in_specs=[pl.BlockSpec(memory_space=pltpu.ANY)],
        out_specs=pl.BlockSpec(memory_space=pltpu.ANY),
        scratch_shapes=[pltpu.SemaphoreType.DMA, pltpu.SemaphoreType.DMA],
        compiler_params=pltpu.CompilerParams(collective_id=0),
    )(partial)


ROWS = 256


def _compute(p0, p1, resid, gamma2d):
    def body(a_ref, b_ref, r_ref, g_ref, o_ref):
        y = a_ref[...] + b_ref[...] + r_ref[...]
        rms = jnp.sqrt(jnp.mean(y * y, axis=-1, keepdims=True) + EPS)
        o_ref[...] = y / rms * g_ref[...]

    return pl.pallas_call(
        body,
        grid=(M // ROWS,),
        in_specs=[
            pl.BlockSpec((ROWS, D), lambda i: (i, 0)),
            pl.BlockSpec((ROWS, D), lambda i: (i, 0)),
            pl.BlockSpec((ROWS, D), lambda i: (i, 0)),
            pl.BlockSpec((1, D), lambda i: (0, 0)),
        ],
        out_specs=pl.BlockSpec((ROWS, D), lambda i: (i, 0)),
        out_shape=jax.ShapeDtypeStruct((M, D), jnp.float32),
    )(p0, p1, resid, gamma2d)


def kernel(partial, resid, gamma):
    other = _exchange(partial)
    p0 = partial.reshape(M, D)
    p1 = other.reshape(M, D)
    return _compute(p0, p1, resid, gamma.reshape(1, D))


# baseline (device time: 807032 ns/iter reference)
import jax
import jax.numpy as jnp
from jax import lax
from jax.experimental import pallas as pl
from jax.experimental.pallas import tpu as pltpu

M, D = 8192, 2048
EPS = 1e-6


def _exchange(partial):

    def body(p_ref, out_ref, send_sem, recv_sem):
        my_x = lax.axis_index("x")
        my_y = lax.axis_index("y")
        my_z = lax.axis_index("z")
        nbr = (my_x, 1 - my_y, my_z)

        barrier = pltpu.get_barrier_semaphore()
        pl.semaphore_signal(
            barrier, inc=1, device_id=nbr, device_id_type=pl.DeviceIdType.MESH
        )
        pl.semaphore_wait(barrier, 1)

        rdma = pltpu.make_async_remote_copy(
            src_ref=p_ref,
            dst_ref=out_ref,
            send_sem=send_sem,
            recv_sem=recv_sem,
            device_id=nbr,
            device_id_type=pl.DeviceIdType.MESH,
        )
        rdma.start()
        rdma.wait()

    return pl.pallas_call(
        body,
        out_shape=jax.ShapeDtypeStruct((1, M, D), jnp.float32),
        in_specs=[pl.BlockSpec(memory_space=pl.ANY)],
        out_specs=pl.BlockSpec(memory_space=pl.ANY),
        scratch_shapes=[pltpu.SemaphoreType.DMA, pltpu.SemaphoreType.DMA],
        compiler_params=pltpu.CompilerParams(collective_id=0),
    )(partial)


ROWS = 256


def _compute(p0, p1, resid, gamma2d):
    def body(a_ref, b_ref, r_ref, g_ref, o_ref):
        y = a_ref[...] + b_ref[...] + r_ref[...]
        rms = jnp.sqrt(jnp.mean(y * y, axis=-1, keepdims=True) + EPS)
        o_ref[...] = y / rms * g_ref[...]

    return pl.pallas_call(
        body,
        grid=(M // ROWS,),
        in_specs=[
            pl.BlockSpec((ROWS, D), lambda i: (i, 0)),
            pl.BlockSpec((ROWS, D), lambda i: (i, 0)),
            pl.BlockSpec((ROWS, D), lambda i: (i, 0)),
            pl.BlockSpec((1, D), lambda i: (0, 0)),
        ],
        out_specs=pl.BlockSpec((ROWS, D), lambda i: (i, 0)),
        out_shape=jax.ShapeDtypeStruct((M, D), jnp.float32),
    )(p0, p1, resid, gamma2d)


def kernel(partial, resid, gamma):
    other = _exchange(partial)
    p0 = partial.reshape(M, D)
    p1 = other.reshape(M, D)
    return _compute(p0, p1, resid, gamma.reshape(1, D))


# device time: 777438 ns/iter; 1.0381x vs baseline; 1.0381x over previous
import jax
import jax.numpy as jnp
from jax import lax
from jax.experimental import pallas as pl
from jax.experimental.pallas import tpu as pltpu

M, D = 8192, 2048
EPS = 1e-6

K = 16
ROWS = M // K


def kernel(partial, resid, gamma):
    p2d = partial.reshape(M, D)
    gamma2d = gamma.reshape(1, D)

    def body(
        p_any,
        p_blk,
        r_blk,
        g_blk,
        o_blk,
        recv_hbm,
        stage,
        send_sems,
        recv_sems,
        copy_sem,
    ):
        k = pl.program_id(0)
        my_x = lax.axis_index("x")
        my_y = lax.axis_index("y")
        my_z = lax.axis_index("z")
        nbr = (my_x, 1 - my_y, my_z)

        def chunk_rdma(h):
            return pltpu.make_async_remote_copy(
                src_ref=p_any.at[0, pl.ds(h * ROWS, ROWS), :],
                dst_ref=recv_hbm.at[pl.ds(h * ROWS, ROWS), :],
                send_sem=send_sems.at[h],
                recv_sem=recv_sems.at[h],
                device_id=nbr,
                device_id_type=pl.DeviceIdType.MESH,
            )

        @pl.when(k == 0)
        def _():
            barrier = pltpu.get_barrier_semaphore()
            pl.semaphore_signal(
                barrier, inc=1, device_id=nbr,
                device_id_type=pl.DeviceIdType.MESH,
            )
            pl.semaphore_wait(barrier, 1)
            for h in range(K):
                chunk_rdma(h).start()

        pltpu.make_async_remote_copy(
            src_ref=p_any.at[0, pl.ds(0, ROWS), :],
            dst_ref=recv_hbm.at[pl.ds(k * ROWS, ROWS), :],
            send_sem=send_sems.at[0],
            recv_sem=recv_sems.at[k],
            device_id=nbr,
            device_id_type=pl.DeviceIdType.MESH,
        ).wait_recv()

        cp = pltpu.make_async_copy(
            recv_hbm.at[pl.ds(k * ROWS, ROWS), :], stage, copy_sem
        )
        cp.start()
        cp.wait()

        y = p_blk[...] + stage[...] + r_blk[...]
        rms = jnp.sqrt(jnp.mean(y * y, axis=-1, keepdims=True) + EPS)
        o_blk[...] = y / rms * g_blk[...]

        @pl.when(k == K - 1)
        def _():
            for h in range(K):
                chunk_rdma(h).wait_send()

    return pl.pallas_call(
        body,
        grid=(K,),
        in_specs=[
            pl.BlockSpec(memory_space=pl.ANY),
            pl.BlockSpec((ROWS, D), lambda i: (i, 0)),
            pl.BlockSpec((ROWS, D), lambda i: (i, 0)),
            pl.BlockSpec((1, D), lambda i: (0, 0)),
        ],
        out_specs=[
            pl.BlockSpec((ROWS, D), lambda i: (i, 0)),
            pl.BlockSpec(memory_space=pl.ANY),
        ],
        out_shape=[
            jax.ShapeDtypeStruct((M, D), jnp.float32),
            jax.ShapeDtypeStruct((M, D), jnp.float32),
        ],
        scratch_shapes=[
            pltpu.VMEM((ROWS, D), jnp.float32),
            pltpu.SemaphoreType.DMA((K,)),
            pltpu.SemaphoreType.DMA((K,)),
            pltpu.SemaphoreType.DMA,
        ],
        compiler_params=pltpu.CompilerParams(
            collective_id=0, vmem_limit_bytes=60 * 1024 * 1024
        ),
    )(partial, p2d, resid, gamma2d)[0]


# device time: 508610 ns/iter; 1.5867x vs baseline; 1.5286x over previous
import jax
import jax.numpy as jnp
from jax import lax
from jax.experimental import pallas as pl
from jax.experimental.pallas import tpu as pltpu

M, D = 8192, 2048
EPS = 1e-6

K = 32
C = K // 2
ROWS = M // K
HALF = M // 2


def kernel(partial, resid, gamma):
    p2d = partial.reshape(M, D)
    gamma2d = gamma.reshape(1, D)

    def body(
        p_any,
        p_blk,
        r_blk,
        g_blk,
        o_blk,
        recv_any,
        stage,
        y_send_sems,
        y_recv_sems,
        x_send_sems,
        x_recv_sems,
        copy_sem,
    ):
        k = pl.program_id(0)
        my_x = lax.axis_index("x")
        my_y = lax.axis_index("y")
        my_z = lax.axis_index("z")
        ynbr = (my_x, 1 - my_y, my_z)
        xnbr = (1 - my_x, my_y, my_z)
        half0 = my_x * HALF

        def y_rdma(c):
            return pltpu.make_async_remote_copy(
                src_ref=p_any.at[0, pl.ds(half0 + c * ROWS, ROWS), :],
                dst_ref=recv_any.at[pl.ds(half0 + c * ROWS, ROWS), :],
                send_sem=y_send_sems.at[c],
                recv_sem=y_recv_sems.at[c],
                device_id=ynbr,
                device_id_type=pl.DeviceIdType.MESH,
            )

        def fwd_rdma(c):
            return pltpu.make_async_remote_copy(
                src_ref=recv_any.at[pl.ds(half0 + c * ROWS, ROWS), :],
                dst_ref=recv_any.at[pl.ds(half0 + c * ROWS, ROWS), :],
                send_sem=x_send_sems.at[c],
                recv_sem=x_recv_sems.at[c],
                device_id=xnbr,
                device_id_type=pl.DeviceIdType.MESH,
            )

        @pl.when(k == 0)
        def _():
            barrier = pltpu.get_barrier_semaphore()
            for nbr in (ynbr, xnbr):
                pl.semaphore_signal(
                    barrier, inc=1, device_id=nbr,
                    device_id_type=pl.DeviceIdType.MESH,
                )
            pl.semaphore_wait(barrier, 2)
            for c in range(C):
                y_rdma(c).start()

        @pl.when(k < C)
        def _():
            pltpu.make_async_remote_copy(
                src_ref=p_any.at[0, pl.ds(0, ROWS), :],
                dst_ref=recv_any.at[pl.ds(half0 + k * ROWS, ROWS), :],
                send_sem=y_send_sems.at[0],
                recv_sem=y_recv_sems.at[k],
                device_id=ynbr,
                device_id_type=pl.DeviceIdType.MESH,
            ).wait_recv()
            fwd_rdma(k).start()

        @pl.when(((my_x == 0) & (k >= C)) | ((my_x == 1) & (k < C)))
        def _():
            cc = k % C
            pltpu.make_async_remote_copy(
                src_ref=recv_any.at[pl.ds(0, ROWS), :],
                dst_ref=recv_any.at[
                    pl.ds((1 - my_x) * HALF + cc * ROWS, ROWS), :
                ],
                send_sem=x_send_sems.at[0],
                recv_sem=x_recv_sems.at[cc],
                device_id=xnbr,
                device_id_type=pl.DeviceIdType.MESH,
            ).wait_recv()

        cp = pltpu.make_async_copy(
            recv_any.at[pl.ds(k * ROWS, ROWS), :], stage, copy_sem
        )
        cp.start()
        cp.wait()

        y = p_blk[...] + stage[...] + r_blk[...]
        rms = jnp.sqrt(jnp.mean(y * y, axis=-1, keepdims=True) + EPS)
        o_blk[...] = y / rms * g_blk[...]

        @pl.when(k == K - 1)
        def _():
            for c in range(C):
                y_rdma(c).wait_send()
                fwd_rdma(c).wait_send()

    return pl.pallas_call(
        body,
        grid=(K,),
        in_specs=[
            pl.BlockSpec(memory_space=pl.ANY),
            pl.BlockSpec((ROWS, D), lambda i: (i, 0)),
            pl.BlockSpec((ROWS, D), lambda i: (i, 0)),
            pl.BlockSpec((1, D), lambda i: (0, 0)),
        ],
        out_specs=[
            pl.BlockSpec((ROWS, D), lambda i: (i, 0)),
            pl.BlockSpec(memory_space=pl.ANY),
        ],
        out_shape=[
            jax.ShapeDtypeStruct((M, D), jnp.float32),
            jax.ShapeDtypeStruct((M, D), jnp.float32),
        ],
        scratch_shapes=[
            pltpu.VMEM((ROWS, D), jnp.float32),
            pltpu.SemaphoreType.DMA((C,)),
            pltpu.SemaphoreType.DMA((C,)),
            pltpu.SemaphoreType.DMA((C,)),
            pltpu.SemaphoreType.DMA((C,)),
            pltpu.SemaphoreType.DMA,
        ],
        compiler_params=pltpu.CompilerParams(
            collective_id=0, vmem_limit_bytes=60 * 1024 * 1024
        ),
    )(partial, p2d, resid, gamma2d)[0]
